# baseline (speedup 1.0000x reference)
import functools

import jax
import jax.numpy as jnp
from jax import lax
from jax.experimental import pallas as pl
from jax.experimental.pallas import tpu as pltpu
from jax.experimental.pallas import tpu_sc as plsc

DIM = 64
PDIM = 128
SEQ = 50
HBATCH = 128
SKEW = DIM + 1
NC, NS = 2, 16
NW = NC * NS
LANES = 16


@functools.lru_cache(maxsize=None)
def _make_gather(n_batch, n_table_rows):
    b_per_w = n_batch // NW
    idx_per_w = b_per_w * SEQ
    n_h = b_per_w // HBATCH
    assert b_per_w * NW == n_batch
    assert n_h * HBATCH == b_per_w and n_h >= 4 and n_h % 2 == 0

    mesh = plsc.VectorSubcoreMesh(core_axis_name="c", subcore_axis_name="s")

    @functools.partial(
        pl.kernel,
        out_type=jax.ShapeDtypeStruct((SEQ, DIM, n_batch), jnp.float32),
        mesh=mesh,
        scratch_types=[
            pltpu.VMEM((idx_per_w,), jnp.int32),
            pltpu.VMEM((HBATCH,), jnp.int32),
            pltpu.VMEM((HBATCH,), jnp.int32),
            pltpu.VMEM((HBATCH, PDIM), jnp.float32),
            pltpu.VMEM((HBATCH, PDIM), jnp.float32),
            pltpu.VMEM((HBATCH * SKEW,), jnp.float32),
            pltpu.VMEM((DIM, HBATCH), jnp.float32),
            pltpu.VMEM((DIM, HBATCH), jnp.float32),
            pltpu.SemaphoreType.DMA,
            pltpu.SemaphoreType.DMA,
            pltpu.SemaphoreType.DMA,
            pltpu.SemaphoreType.DMA,
        ],
        compiler_params=pltpu.CompilerParams(needs_layout_passes=False),
    )
    def gather_kernel(table_hbm, idx_hbm, out_hbm, idx_v, idxs0_v, idxs1_v,
                      rows0_v, rows1_v, skew_v, tile0_v, tile1_v,
                      gsem0, gsem1, ssem0, ssem1):
        gsems = (gsem0, gsem1)
        ssems = (ssem0, ssem1)
        idxs_bufs = (idxs0_v, idxs1_v)
        rows_bufs = (rows0_v, rows1_v)
        tile_bufs = (tile0_v, tile1_v)
        wid = lax.axis_index("s") * NC + lax.axis_index("c")
        b0 = wid * b_per_w

        pltpu.sync_copy(idx_hbm.at[pl.ds(wid * idx_per_w, idx_per_w)], idx_v)

        iota = lax.iota(jnp.int32, LANES)
        iota_seq = iota * SEQ
        iota_skew = iota * SKEW
        vbases = [iota_skew + cc * (LANES * SKEW)
                  for cc in range(HBATCH // LANES)]

        def build_idx(s, h, p):
            for c in range(HBATCH // LANES):
                base = (h * HBATCH + c * LANES) * SEQ
                vals = plsc.load_gather(idx_v, [iota_seq + (base + s)])
                idxs_bufs[p][pl.ds(c * LANES, LANES)] = vals

        def gather(p):
            pltpu.async_copy(
                table_hbm.at[idxs_bufs[p]], rows_bufs[p], gsems[p])

        def wait_gather(p):
            pltpu.make_async_copy(
                table_hbm.at[pl.ds(0, HBATCH)], rows_bufs[p],
                gsems[p]).wait()

        def transpose(p):
            rows = rows_bufs[p]
            tile = tile_bufs[p]

            @pl.loop(0, HBATCH, unroll=2)
            def _(j):
                jbase = j * SKEW
                for c in range(DIM // LANES):
                    skew_v[pl.ds(jbase + c * LANES, LANES)] = (
                        rows[j, pl.ds(c * LANES, LANES)])

            @pl.loop(0, DIM, unroll=4)
            def _(d):
                for cc in range(HBATCH // LANES):
                    v = plsc.load_gather(skew_v, [vbases[cc] + d])
                    tile[d, pl.ds(cc * LANES, LANES)] = v

        def store(s, h, p):
            pltpu.async_copy(
                tile_bufs[p],
                out_hbm.at[s, pl.ds(0, DIM), pl.ds(b0 + h * HBATCH, HBATCH)],
                ssems[p])

        def wait_store(p):
            pltpu.make_async_copy(
                tile_bufs[p],
                out_hbm.at[0, pl.ds(0, DIM), pl.ds(0, HBATCH)],
                ssems[p]).wait()

        build_idx(0, 0, 0)
        gather(0)
        build_idx(0, 1, 1)
        gather(1)

        @pl.loop(0, SEQ)
        def _(s):
            for h in range(n_h):
                p = h % 2
                wait_gather(p)
                if h < 2:

                    @pl.when(s > 0)
                    def _():
                        wait_store(p)

                else:
                    wait_store(p)
                if h < n_h - 2:
                    build_idx(s, h + 2, p)
                    transpose(p)
                    gather(p)
                else:

                    @pl.when(s < SEQ - 1)
                    def _():
                        build_idx(s + 1, h - (n_h - 2), p)

                    transpose(p)

                    @pl.when(s < SEQ - 1)
                    def _():
                        gather(p)

                store(s, h, p)

        wait_store(0)
        wait_store(1)

    return gather_kernel


def kernel(node_positions, psne_layer):
    b, s = node_positions.shape
    idx_flat = node_positions.reshape(b * s).astype(jnp.int32)
    table128 = jnp.pad(psne_layer, ((0, 0), (0, PDIM - DIM)))
    fn = _make_gather(b, psne_layer.shape[0])
    out_t = fn(table128, idx_flat)
    return out_t.transpose(2, 0, 1)


# speedup vs baseline: 1.1184x; 1.1184x over previous
import functools

import jax
import jax.numpy as jnp
from jax import lax
from jax.experimental import pallas as pl
from jax.experimental.pallas import tpu as pltpu
from jax.experimental.pallas import tpu_sc as plsc

DIM = 64
PDIM = 128
SEQ = 50
GB = 4
HB = GB // 2
NC, NS = 2, 16
NW = NC * NS


@functools.lru_cache(maxsize=None)
def _make_gather(n_batch, n_table_rows):
    b_per_w = n_batch // NW
    n_groups = b_per_w // GB
    idx_per_w = b_per_w * SEQ
    grows = GB * SEQ
    assert b_per_w * NW == n_batch
    assert n_groups * GB == b_per_w
    assert n_groups % 2 == 0 and grows % 8 == 0

    mesh = plsc.VectorSubcoreMesh(core_axis_name="c", subcore_axis_name="s")

    @functools.partial(
        pl.kernel,
        out_type=jax.ShapeDtypeStruct((n_batch, SEQ, DIM), jnp.float32),
        mesh=mesh,
        scratch_types=[
            pltpu.VMEM((idx_per_w,), jnp.int32),
            pltpu.VMEM((2, grows, PDIM), jnp.float32),
            pltpu.VMEM((2, HB, SEQ, DIM), jnp.float32),
            pltpu.SemaphoreType.DMA,
            pltpu.SemaphoreType.DMA,
            pltpu.SemaphoreType.DMA,
            pltpu.SemaphoreType.DMA,
        ],
    )
    def gather_kernel(table_hbm, idx_hbm, out_hbm, idx_v, rows_v, pack_v,
                      gsem0, gsem1, psem0, psem1):
        gsems = (gsem0, gsem1)
        psems = (psem0, psem1)
        wid = lax.axis_index("s") * NC + lax.axis_index("c")
        b0 = wid * b_per_w

        pltpu.sync_copy(idx_hbm.at[pl.ds(wid * idx_per_w, idx_per_w)], idx_v)

        def gather(g, buf):
            pltpu.async_copy(
                table_hbm.at[idx_v.at[pl.ds(g * grows, grows)]],
                rows_v.at[buf], gsems[buf])

        def wait_gather(buf):
            pltpu.make_async_copy(
                table_hbm.at[pl.ds(0, grows)], rows_v.at[buf],
                gsems[buf]).wait()

        def repack(buf, h):
            for i in range(HB):
                base = (h * HB + i) * SEQ

                @pl.loop(0, SEQ, unroll=2)
                def _(s):
                    for c in range(DIM // 16):
                        pack_v[h, i, s, pl.ds(c * 16, 16)] = (
                            rows_v[buf, base + s, pl.ds(c * 16, 16)])

        def store(g, h):
            pltpu.async_copy(
                pack_v.at[h],
                out_hbm.at[pl.ds(b0 + g * GB + h * HB, HB)],
                psems[h])

        def wait_store(h):
            pltpu.make_async_copy(
                pack_v.at[h], out_hbm.at[pl.ds(b0, HB)], psems[h]).wait()

        gather(0, 0)
        gather(1, 1)

        @pl.loop(0, n_groups, step=2)
        def _(j0):
            for buf in range(2):
                j = j0 + buf
                wait_gather(buf)
                for h in range(2):

                    @pl.when(j > 0)
                    def _():
                        wait_store(h)

                    repack(buf, h)
                    store(j, h)

                @pl.when(j < n_groups - 2)
                def _():
                    gather(j + 2, buf)

        wait_store(0)
        wait_store(1)

    return gather_kernel


NCHUNK = 4


def kernel(node_positions, psne_layer):
    b, s = node_positions.shape
    idx_flat = node_positions.reshape(b * s).astype(jnp.int32)
    table128 = jnp.pad(psne_layer, ((0, 0), (0, PDIM - DIM)))
    nb = b // NCHUNK
    fn = _make_gather(nb, psne_layer.shape[0])
    outs = [fn(table128, lax.slice(idx_flat, (k * nb * s,),
                                   ((k + 1) * nb * s,)))
            for k in range(NCHUNK)]
    return jnp.concatenate(outs, axis=0)
